# Initial kernel scaffold; baseline (speedup 1.0000x reference)
import jax
import jax.numpy as jnp
from jax.experimental import pallas as pl


def kernel(user_emb, paper_emb, er, ew):
    raise NotImplementedError("write your pallas kernel here")


# speedup vs baseline: 7.3437x; 7.3437x over previous
import jax
import jax.numpy as jnp
from jax import lax
from jax.experimental import pallas as pl
from jax.experimental.pallas import tpu as pltpu
from jax.experimental.pallas import tpu_sc as plsc

NU = 100000
E = 1600000
NT = 16
NR = 100096
RPT = NR // NT
CR = 272
NKC = RPT // CR
EK = 512
EP = 1605632
NCH = EP // (NT * EK)
ECR = EP // 128
CPT = ECR // NT


def _sc_body(ut, pt, er3, ew3,
             usum, psum, xu, xp, tmp, invt,
             sidx, gidx, msg, ones, zc1, cnvb, invb, sem,
             acc, cnt_a, cnt_b):
  cid = lax.axis_index("c")
  sid = lax.axis_index("s")
  goff = cid * NR
  tb = sid * RPT
  eb = sid * CPT

  zero16 = jnp.zeros((16,), jnp.float32)
  one16 = jnp.ones((16,), jnp.float32)

  def _init_zc(i, _):
    zc1[pl.ds(i * 16, 16)] = zero16
    return _
  lax.fori_loop(0, CR // 16, _init_zc, None)

  def _init_o(i, _):
    ones[pl.ds(i * 16, 16)] = one16
    return _
  lax.fori_loop(0, 8, _init_o, None)

  def bincount(rel3, side, cnt, tno):
    def _zk(k, _):
      rb = pl.multiple_of(tb + k * CR, 16)
      pltpu.sync_copy(zc1, cnt.at[pl.ds(rb, CR)])
      return _
    lax.fori_loop(0, NKC, _zk, None)
    plsc.subcore_barrier()

    def chunk(c, _):
      pltpu.sync_copy(rel3.at[side, pl.ds(eb + c * 4, 4), :], sidx)
      cps = [pltpu.async_copy(ones, cnt.at[sidx.at[j]], sem, add=True)
             for j in range(4)]
      for cp in cps:
        cp.wait()
      return _
    lax.fori_loop(0, NCH, chunk, None)
    plsc.subcore_barrier()

    def _ck(k, _):
      rb = pl.multiple_of(tb + k * CR, 16)
      pltpu.sync_copy(cnt.at[pl.ds(rb, CR)], cnvb)

      def conv(i, _):
        x = cnvb[pl.ds(i * 16, 16)]
        cnvb[pl.ds(i * 16, 16)] = 0.5 / jnp.maximum(x, 1.0)
        return _
      lax.fori_loop(0, CR // 16, conv, None)
      pltpu.sync_copy(cnvb, invt.at[cid, tno, pl.ds(rb, CR)])
      return _
    lax.fori_loop(0, NKC, _ck, None)
    plsc.subcore_barrier()

  bincount(er3, 0, cnt_a, 0)
  bincount(ew3, 0, cnt_b, 1)
  bincount(er3, 1, cnt_a, 2)
  bincount(ew3, 1, cnt_b, 3)

  if True:
    def zero_acc():
      def _z(i, _):
        msg[i, :] = zero16
        return _
      lax.fori_loop(0, CR, _z, None)

      def _zk(k, _):
        rb = pl.multiple_of(tb + k * CR, 16)
        pltpu.sync_copy(msg.at[pl.ds(0, CR), :], acc.at[pl.ds(rb, CR), :])
        return _
      lax.fori_loop(0, NKC, _zk, None)
      plsc.subcore_barrier()

    def scatter_pass(rel3, gside, sside, gtab):
      def chunk(c, _):
        pltpu.sync_copy(rel3.at[gside, pl.ds(eb + c * 4, 4), :], gidx)
        pltpu.sync_copy(rel3.at[sside, pl.ds(eb + c * 4, 4), :], sidx)

        def adj(i, _):
          for j in range(4):
            gidx[j, pl.ds(i * 16, 16)] = gidx[j, pl.ds(i * 16, 16)] + goff
          return _
        lax.fori_loop(0, 8, adj, None)

        cps = [pltpu.async_copy(gtab.at[gidx.at[j]],
                                msg.at[pl.ds(j * 128, 128), :], sem)
               for j in range(4)]
        for cp in cps:
          cp.wait()
        for j in range(4):
          pltpu.sync_copy(msg.at[pl.ds(j * 128, 128), :],
                          acc.at[sidx.at[j]], add=True)
        return _
      lax.fori_loop(0, NCH, chunk, None)
      plsc.subcore_barrier()

    def p1_writeback(tno):
      def _pk(k, _):
        rb = pl.multiple_of(tb + k * CR, 16)
        pltpu.sync_copy(acc.at[pl.ds(rb, CR), :], msg.at[pl.ds(0, CR), :])
        pltpu.sync_copy(invt.at[cid, tno, pl.ds(rb, CR)], invb)

        def rows(i, _):
          iv = invb[pl.ds(i * 16, 16)]
          for u in range(16):
            r = i * 16 + u
            msg[r, :] = msg[r, :] * iv[u]
          return _
        lax.fori_loop(0, CR // 16, rows, None)
        pltpu.sync_copy(msg.at[pl.ds(0, CR), :], tmp.at[pl.ds(goff + rb, CR), :])
        return _
      lax.fori_loop(0, NKC, _pk, None)
      plsc.subcore_barrier()

    def combine(tno, out_tab, sum_out, init_tab, layer):
      def _ck(k, _):
        rb = pl.multiple_of(tb + k * CR, 16)
        pltpu.sync_copy(acc.at[pl.ds(rb, CR), :], msg.at[pl.ds(0, CR), :])
        pltpu.sync_copy(invt.at[cid, tno, pl.ds(rb, CR)], invb)
        pltpu.sync_copy(tmp.at[pl.ds(goff + rb, CR), :], msg.at[pl.ds(CR, CR), :])
        if layer == 0:
          pltpu.sync_copy(init_tab.at[pl.ds(goff + rb, CR), :],
                          msg.at[pl.ds(2 * CR, CR), :])
        else:
          pltpu.sync_copy(sum_out.at[cid, pl.ds(rb, CR), :],
                          msg.at[pl.ds(2 * CR, CR), :])

        def rows(i, _):
          iv = invb[pl.ds(i * 16, 16)]
          for u in range(16):
            r = i * 16 + u
            xrow = msg[CR + r, :] + msg[r, :] * iv[u]
            srow = msg[2 * CR + r, :] + xrow
            if layer == 2:
              srow = srow * 0.25
            msg[CR + r, :] = xrow
            msg[2 * CR + r, :] = srow
          return _
        lax.fori_loop(0, CR // 16, rows, None)
        pltpu.sync_copy(msg.at[pl.ds(CR, CR), :],
                        out_tab.at[pl.ds(goff + rb, CR), :])
        pltpu.sync_copy(msg.at[pl.ds(2 * CR, CR), :],
                        sum_out.at[cid, pl.ds(rb, CR), :])
        return _
      lax.fori_loop(0, NKC, _ck, None)
      plsc.subcore_barrier()

    for layer in range(3):
      gt_u = pt if layer == 0 else xp
      zero_acc()
      scatter_pass(er3, 1, 0, gt_u)
      p1_writeback(0)
      zero_acc()
      scatter_pass(ew3, 1, 0, gt_u)
      combine(1, xu, usum, ut, layer)
      zero_acc()
      scatter_pass(er3, 0, 1, xu)
      p1_writeback(2)
      zero_acc()
      scatter_pass(ew3, 0, 1, xu)
      combine(3, xp, psum, pt, layer)


@jax.jit
def _run(ut, pt, er3, ew3):
  f32 = jnp.float32
  mesh = plsc.VectorSubcoreMesh(core_axis_name="c", subcore_axis_name="s",
                                num_cores=2, num_subcores=16)
  out = pl.kernel(
      _sc_body,
      out_type=(
          jax.ShapeDtypeStruct((2, NR, 16), f32),
          jax.ShapeDtypeStruct((2, NR, 16), f32),
          jax.ShapeDtypeStruct((2 * NR, 16), f32),
          jax.ShapeDtypeStruct((2 * NR, 16), f32),
          jax.ShapeDtypeStruct((2 * NR, 16), f32),
          jax.ShapeDtypeStruct((2, 4, NR), f32),
      ),
      mesh=mesh,
      compiler_params=pltpu.CompilerParams(use_tc_tiling_on_sc=False),
      scratch_types=(
          pltpu.VMEM((4, 128), jnp.int32),
          pltpu.VMEM((4, 128), jnp.int32),
          pltpu.VMEM((3 * CR, 16), f32),
          pltpu.VMEM((128,), f32),
          pltpu.VMEM((CR,), f32),
          pltpu.VMEM((CR,), f32),
          pltpu.VMEM((CR,), f32),
          pltpu.SemaphoreType.DMA,
          pltpu.VMEM_SHARED((NR, 16), f32),
          pltpu.VMEM_SHARED((NR,), f32),
          pltpu.VMEM_SHARED((NR,), f32),
      ),
  )(ut, pt, er3, ew3)
  return out[0], out[1]


def kernel(user_emb, paper_emb, er, ew):
  pad = jnp.full((2, EP - E), NU, jnp.int32)
  er3 = jnp.concatenate([er, pad], axis=1).reshape(2, ECR, 128)
  ew3 = jnp.concatenate([ew, pad], axis=1).reshape(2, ECR, 128)
  zrow = jnp.zeros((NR - NU, 16), jnp.float32)
  ut = jnp.concatenate([user_emb[:, :16], zrow, user_emb[:, 16:], zrow], axis=0)
  pt = jnp.concatenate([paper_emb[:, :16], zrow, paper_emb[:, 16:], zrow], axis=0)
  usum, psum = _run(ut, pt, er3, ew3)
  u = jnp.concatenate([usum[0, :NU], usum[1, :NU]], axis=1)
  p = jnp.concatenate([psum[0, :NU], psum[1, :NU]], axis=1)
  return u, p
